# baseline (speedup 1.0000x reference)
import jax
import jax.numpy as jnp
from jax import lax
from jax.experimental import pallas as pl
from jax.experimental.pallas import tpu as pltpu
from jax.experimental.pallas import tpu_sc as plsc

_VOCAB = 12
_NC = 2
_NS = 16
_NW = _NC * _NS
_CHUNK = 2048
_NPAIR = _VOCAB * _VOCAB
_PROW = 2 * _VOCAB
_OUTW = _CHUNK * _VOCAB


def _ptab_body(e_ref, p_ref, b_ref, t_ref):
    v = _VOCAB
    t = lax.dot_general(
        e_ref[...], p_ref[...], (((1,), (1,)), ((), ())),
        preferred_element_type=jnp.float32,
    ) + b_ref[...]
    r_a = jnp.broadcast_to(t[:, None, :], (v, v, v)).reshape(_NPAIR, v)
    r_b = jnp.broadcast_to(t[None, :, :], (v, v, v)).reshape(_NPAIR, v)
    t_ref[...] = jnp.concatenate([r_a, r_b], axis=1)


def _pair_table(embed_weight, proj_weight, proj_bias):
    return pl.pallas_call(
        _ptab_body,
        out_shape=jax.ShapeDtypeStruct((_NPAIR, _PROW), jnp.float32),
    )(embed_weight, proj_weight, proj_bias.reshape(1, _VOCAB))


def _make_lookup(n_tokens):
    per_w = n_tokens // _NW
    nchunks = per_w // _CHUNK
    assert per_w * _NW == n_tokens and nchunks * _CHUNK == per_w
    assert nchunks % 2 == 0

    mesh = plsc.VectorSubcoreMesh(
        core_axis_name="c", subcore_axis_name="s",
        num_cores=_NC, num_subcores=_NS,
    )

    def body(ids_hbm, ptab_hbm, out_hbm,
             idx0, idx1, pair_v, out0, out1, ptab_v,
             sem_i0, sem_i1, sem_o0, sem_o1):
        wid = lax.axis_index("s") * _NC + lax.axis_index("c")
        tok0 = wid * per_w
        idx_bufs = (idx0, idx1)
        out_bufs = (out0, out1)
        sem_i = (sem_i0, sem_i1)
        sem_o = (sem_o0, sem_o1)

        pltpu.sync_copy(ptab_hbm, ptab_v)

        iota = lax.iota(jnp.int32, 16)
        sel = (
            iota * 0,
            (iota >= 8).astype(jnp.int32),
            iota * 0 + 1,
        )
        off = (
            iota,
            jnp.where(iota < 8, iota + 16, iota - 8),
            iota + 8,
        )

        def ids_copy(b, c):
            return pltpu.make_async_copy(
                ids_hbm.at[pl.ds(tok0 + c * _CHUNK, _CHUNK)],
                idx_bufs[b], sem_i[b],
            )

        def out_copy(b, c):
            return pltpu.make_async_copy(
                out_bufs[b],
                out_hbm.at[pl.ds((tok0 + c * _CHUNK) * _VOCAB, _OUTW)],
                sem_o[b],
            )

        ids_copy(0, 0).start()
        ids_copy(1, 1).start()

        def superstep(k, carry):
            for b in (0, 1):
                c = k * 2 + b
                idx_v = idx_bufs[b]
                out_v = out_bufs[b]
                ids_copy(b, c).wait()

                def pstep(i, carry2):
                    t0 = i * 32 + iota * 2
                    pa = plsc.load_gather(idx_v, [t0])
                    pb = plsc.load_gather(idx_v, [t0 + 1])
                    pair_v[pl.ds(i * 16, 16)] = pa * _VOCAB + pb
                    return carry2

                lax.fori_loop(0, _CHUNK // 32, pstep, 0)

                @pl.when(k < nchunks // 2 - 1)
                def _prefetch():
                    ids_copy(b, c + 2).start()

                @pl.when(k > 0)
                def _drain():
                    out_copy(b, c).wait()

                def mstep(g, carry2):
                    pbase = g * 2
                    wbase = g * 48
                    for r in range(3):
                        p = plsc.load_gather(pair_v, [pbase + sel[r]])
                        val = plsc.load_gather(ptab_v, [p, off[r]])
                        out_v[pl.ds(wbase + r * 16, 16)] = val
                    return carry2

                lax.fori_loop(0, _OUTW // 48, mstep, 0)
                out_copy(b, c).start()
            return carry

        lax.fori_loop(0, nchunks // 2, superstep, 0)
        out_copy(0, nchunks - 2).wait()
        out_copy(1, nchunks - 1).wait()

    return pl.kernel(
        body,
        out_type=jax.ShapeDtypeStruct((n_tokens * _VOCAB,), jnp.float32),
        mesh=mesh,
        scratch_types=[
            pltpu.VMEM((_CHUNK,), jnp.int32),
            pltpu.VMEM((_CHUNK,), jnp.int32),
            pltpu.VMEM((_CHUNK // 2,), jnp.int32),
            pltpu.VMEM((_OUTW,), jnp.float32),
            pltpu.VMEM((_OUTW,), jnp.float32),
            pltpu.VMEM((_NPAIR, _PROW), jnp.float32),
            pltpu.SemaphoreType.DMA,
            pltpu.SemaphoreType.DMA,
            pltpu.SemaphoreType.DMA,
            pltpu.SemaphoreType.DMA,
        ],
        compiler_params=pltpu.CompilerParams(
            use_tc_tiling_on_sc=False, needs_layout_passes=False,
        ),
    )


@jax.jit
def kernel(input_ids, embed_weight, proj_weight, proj_bias):
    b, s = input_ids.shape
    n = b * s
    ptab = _pair_table(embed_weight, proj_weight, proj_bias)
    out = _make_lookup(n)(input_ids.reshape(n), ptab)
    return out.reshape(b, s, _VOCAB)


# speedup vs baseline: 1.2938x; 1.2938x over previous
import jax
import jax.numpy as jnp
from jax import lax
from jax.experimental import pallas as pl
from jax.experimental.pallas import tpu as pltpu
from jax.experimental.pallas import tpu_sc as plsc

_VOCAB = 12
_NC = 2
_NS = 16
_NW = _NC * _NS
_CHUNK = 2048
_NPAIR = _VOCAB * _VOCAB
_PROW = 2 * _VOCAB
_OUTW = _CHUNK * _VOCAB


def _ptab_body(e_ref, p_ref, b_ref, t_ref):
    v = _VOCAB
    t = lax.dot_general(
        e_ref[...], p_ref[...], (((1,), (1,)), ((), ())),
        preferred_element_type=jnp.float32,
    ) + b_ref[...]
    r_a = jnp.broadcast_to(t[:, None, :], (v, v, v)).reshape(_NPAIR, v)
    r_b = jnp.broadcast_to(t[None, :, :], (v, v, v)).reshape(_NPAIR, v)
    t_ref[...] = jnp.concatenate([r_a, r_b], axis=1)


def _pair_table(embed_weight, proj_weight, proj_bias):
    return pl.pallas_call(
        _ptab_body,
        out_shape=jax.ShapeDtypeStruct((_NPAIR, _PROW), jnp.float32),
    )(embed_weight, proj_weight, proj_bias.reshape(1, _VOCAB))


def _make_lookup(n_tokens):
    per_w = n_tokens // _NW
    nchunks = per_w // _CHUNK
    assert per_w * _NW == n_tokens and nchunks * _CHUNK == per_w
    assert nchunks % 2 == 0

    mesh = plsc.VectorSubcoreMesh(
        core_axis_name="c", subcore_axis_name="s",
        num_cores=_NC, num_subcores=_NS,
    )

    def body(ids_hbm, ptab_hbm, out_hbm,
             idx0, idx1, pair_v, out0, out1, ptab_v,
             sem_i0, sem_i1, sem_o0, sem_o1):
        wid = lax.axis_index("s") * _NC + lax.axis_index("c")
        tok0 = wid * per_w
        idx_bufs = (idx0, idx1)
        out_bufs = (out0, out1)
        sem_i = (sem_i0, sem_i1)
        sem_o = (sem_o0, sem_o1)

        pltpu.sync_copy(ptab_hbm, ptab_v)

        iota = lax.iota(jnp.int32, 16)
        sel = (
            iota * 0,
            (iota >= 8).astype(jnp.int32),
            iota * 0 + 1,
        )
        off = (
            iota,
            jnp.where(iota < 8, iota + 16, iota - 8),
            iota + 8,
        )

        def ids_copy(b, c):
            return pltpu.make_async_copy(
                ids_hbm.at[pl.ds(tok0 + c * _CHUNK, _CHUNK)],
                idx_bufs[b], sem_i[b],
            )

        def out_copy(b, c):
            return pltpu.make_async_copy(
                out_bufs[b],
                out_hbm.at[pl.ds((tok0 + c * _CHUNK) * _VOCAB, _OUTW)],
                sem_o[b],
            )

        ids_copy(0, 0).start()
        ids_copy(1, 1).start()

        def superstep(k, carry):
            for b in (0, 1):
                c = k * 2 + b
                idx_v = idx_bufs[b]
                out_v = out_bufs[b]
                ids_copy(b, c).wait()

                @plsc.parallel_loop(0, _CHUNK // 32, unroll=4)
                def pstep(i):
                    t0 = i * 32 + iota * 2
                    pa = plsc.load_gather(idx_v, [t0])
                    pb = plsc.load_gather(idx_v, [t0 + 1])
                    pair_v[pl.ds(i * 16, 16)] = pa * _VOCAB + pb

                @pl.when(k < nchunks // 2 - 1)
                def _prefetch():
                    ids_copy(b, c + 2).start()

                @pl.when(k > 0)
                def _drain():
                    out_copy(b, c).wait()

                @plsc.parallel_loop(0, _OUTW // 48, unroll=4)
                def mstep(g):
                    pbase = g * 2
                    wbase = g * 48
                    for r in range(3):
                        p = plsc.load_gather(pair_v, [pbase + sel[r]])
                        val = plsc.load_gather(ptab_v, [p, off[r]])
                        out_v[pl.ds(wbase + r * 16, 16)] = val
                out_copy(b, c).start()
            return carry

        lax.fori_loop(0, nchunks // 2, superstep, 0)
        out_copy(0, nchunks - 2).wait()
        out_copy(1, nchunks - 1).wait()

    return pl.kernel(
        body,
        out_type=jax.ShapeDtypeStruct((n_tokens * _VOCAB,), jnp.float32),
        mesh=mesh,
        scratch_types=[
            pltpu.VMEM((_CHUNK,), jnp.int32),
            pltpu.VMEM((_CHUNK,), jnp.int32),
            pltpu.VMEM((_CHUNK // 2,), jnp.int32),
            pltpu.VMEM((_OUTW,), jnp.float32),
            pltpu.VMEM((_OUTW,), jnp.float32),
            pltpu.VMEM((_NPAIR, _PROW), jnp.float32),
            pltpu.SemaphoreType.DMA,
            pltpu.SemaphoreType.DMA,
            pltpu.SemaphoreType.DMA,
            pltpu.SemaphoreType.DMA,
        ],
        compiler_params=pltpu.CompilerParams(
            use_tc_tiling_on_sc=False, needs_layout_passes=False,
        ),
    )


@jax.jit
def kernel(input_ids, embed_weight, proj_weight, proj_bias):
    b, s = input_ids.shape
    n = b * s
    ptab = _pair_table(embed_weight, proj_weight, proj_bias)
    out = _make_lookup(n)(input_ids.reshape(n), ptab)
    return out.reshape(b, s, _VOCAB)


# speedup vs baseline: 34.8434x; 26.9303x over previous
import jax
import jax.numpy as jnp
from jax import lax
from jax.experimental import pallas as pl
from jax.experimental.pallas import tpu as pltpu
from jax.experimental.pallas import tpu_sc as plsc

_VOCAB = 12
_NC = 2
_NS = 16
_NW = _NC * _NS
_SROW = 8


def _tab_body(e_ref, p_ref, b_ref, t_ref):
    t_ref[...] = lax.dot_general(
        p_ref[...], e_ref[...], (((1,), (1,)), ((), ())),
        preferred_element_type=jnp.float32,
    ) + b_ref[...]


def _fused_table_t(embed_weight, proj_weight, proj_bias):
    return pl.pallas_call(
        _tab_body,
        out_shape=jax.ShapeDtypeStruct((_VOCAB, _VOCAB), jnp.float32),
    )(embed_weight, proj_weight, proj_bias.reshape(_VOCAB, 1))


def _make_lookup(seq, batch):
    per_b = batch // _NW
    nchunks = seq // _SROW
    assert per_b * _NW == batch and nchunks * _SROW == seq
    assert per_b % 128 == 0
    nvec = _SROW * per_b // 16

    mesh = plsc.VectorSubcoreMesh(
        core_axis_name="c", subcore_axis_name="s",
        num_cores=_NC, num_subcores=_NS,
    )

    def body(ids_hbm, tab_hbm, out_hbm,
             idx0, idx1, out0, out1, tab_v,
             sem_i0, sem_i1, sem_o0, sem_o1):
        wid = lax.axis_index("s") * _NC + lax.axis_index("c")
        b0 = wid * per_b
        idx_bufs = (idx0, idx1)
        out_bufs = (out0, out1)
        sem_i = (sem_i0, sem_i1)
        sem_o = (sem_o0, sem_o1)

        pltpu.sync_copy(tab_hbm, tab_v)

        def ids_copy(b, c):
            return pltpu.make_async_copy(
                ids_hbm.at[pl.ds(c * _SROW, _SROW), pl.ds(b0, per_b)],
                idx_bufs[b], sem_i[b],
            )

        def out_copy(b, c):
            return pltpu.make_async_copy(
                out_bufs[b],
                out_hbm.at[:, pl.ds(c * _SROW, _SROW), pl.ds(b0, per_b)],
                sem_o[b],
            )

        ids_copy(0, 0).start()
        ids_copy(1, 1).start()

        def chunk(b, c, k):
            idx_v = idx_bufs[b]
            out_v = out_bufs[b]
            ids_copy(b, c).wait()

            @pl.when(k < nchunks - 2)
            def _prefetch():
                ids_copy(b, c + 2).start()

            @pl.when(k > 1)
            def _drain():
                out_copy(b, c).wait()

            @plsc.parallel_loop(0, nvec, unroll=2)
            def vstep(j):
                r = j // (per_b // 16)
                col = (j % (per_b // 16)) * 16
                ids16 = idx_v[r, pl.ds(col, 16)]
                for v in range(_VOCAB):
                    out_v[v, r, pl.ds(col, 16)] = plsc.load_gather(
                        tab_v, [ids16 * 0 + v, ids16])
                return None

            out_copy(b, c).start()

        def superstep(k2, carry):
            chunk(0, k2 * 2, k2 * 2)
            chunk(1, k2 * 2 + 1, k2 * 2 + 1)
            return carry

        lax.fori_loop(0, nchunks // 2, superstep, 0)
        if nchunks % 2:
            chunk(0, nchunks - 1, nchunks - 1)
        out_copy(nchunks % 2, nchunks - 2 + (nchunks % 2)).wait()
        out_copy(1 - nchunks % 2, nchunks - 1).wait()

    return pl.kernel(
        body,
        out_type=jax.ShapeDtypeStruct((_VOCAB, seq, batch), jnp.float32),
        mesh=mesh,
        scratch_types=[
            pltpu.VMEM((_SROW, per_b), jnp.int32),
            pltpu.VMEM((_SROW, per_b), jnp.int32),
            pltpu.VMEM((_VOCAB, _SROW, per_b), jnp.float32),
            pltpu.VMEM((_VOCAB, _SROW, per_b), jnp.float32),
            pltpu.VMEM((_VOCAB, _VOCAB), jnp.float32),
            pltpu.SemaphoreType.DMA,
            pltpu.SemaphoreType.DMA,
            pltpu.SemaphoreType.DMA,
            pltpu.SemaphoreType.DMA,
        ],
        compiler_params=pltpu.CompilerParams(
            use_tc_tiling_on_sc=True, needs_layout_passes=False,
        ),
    )


@jax.jit
def kernel(input_ids, embed_weight, proj_weight, proj_bias):
    b, s = input_ids.shape
    tab = _fused_table_t(embed_weight, proj_weight, proj_bias)
    out3 = _make_lookup(s, b)(input_ids.T, tab)
    return out3.transpose(2, 1, 0)


# speedup vs baseline: 40.4302x; 1.1603x over previous
import jax
import jax.numpy as jnp
from jax import lax
from jax.experimental import pallas as pl
from jax.experimental.pallas import tpu as pltpu
from jax.experimental.pallas import tpu_sc as plsc

_VOCAB = 12
_NC = 2
_NS = 16
_NW = _NC * _NS


def _tab_body(e_ref, p_ref, b_ref, t_ref):
    t_ref[...] = lax.dot_general(
        p_ref[...], e_ref[...], (((1,), (1,)), ((), ())),
        preferred_element_type=jnp.float32,
    ) + b_ref[...]


def _fused_table_t(embed_weight, proj_weight, proj_bias):
    return pl.pallas_call(
        _tab_body,
        out_shape=jax.ShapeDtypeStruct((_VOCAB, _VOCAB), jnp.float32),
    )(embed_weight, proj_weight, proj_bias.reshape(_VOCAB, 1))


def _make_lookup(nst, nbt):
    tiles_w = nbt // _NW
    assert tiles_w * _NW == nbt
    nvec = tiles_w * 8 * 128 // 16

    mesh = plsc.VectorSubcoreMesh(
        core_axis_name="c", subcore_axis_name="s",
        num_cores=_NC, num_subcores=_NS,
    )

    def body(ids_hbm, tab_hbm, out_hbm,
             idx0, idx1, out0, out1, tab_v,
             sem_i0, sem_i1, sem_o0, sem_o1):
        wid = lax.axis_index("s") * _NC + lax.axis_index("c")
        bt0 = wid * tiles_w
        idx_bufs = (idx0, idx1)
        out_bufs = (out0, out1)
        sem_i = (sem_i0, sem_i1)
        sem_o = (sem_o0, sem_o1)

        pltpu.sync_copy(tab_hbm, tab_v)

        def ids_copy(b, c):
            return pltpu.make_async_copy(
                ids_hbm.at[c, pl.ds(bt0, tiles_w)],
                idx_bufs[b], sem_i[b],
            )

        def out_copy(b, c):
            return pltpu.make_async_copy(
                out_bufs[b],
                out_hbm.at[:, c, pl.ds(bt0, tiles_w)],
                sem_o[b],
            )

        ids_copy(0, 0).start()
        ids_copy(1, 1).start()

        def chunk(b, c, k):
            idx_v = idx_bufs[b]
            out_v = out_bufs[b]
            ids_copy(b, c).wait()

            @pl.when(k > 1)
            def _drain():
                out_copy(b, c).wait()

            @plsc.parallel_loop(0, nvec, unroll=2)
            def vstep(j):
                t = j // (8 * 8)
                rem = j % (8 * 8)
                r = rem // 8
                col = (rem % 8) * 16
                ids16 = idx_v[t, r, pl.ds(col, 16)]
                for v in range(_VOCAB):
                    out_v[v, t, r, pl.ds(col, 16)] = plsc.load_gather(
                        tab_v, [ids16 + v * _VOCAB])
                return None

            @pl.when(k < nst - 2)
            def _prefetch():
                ids_copy(b, c + 2).start()

            out_copy(b, c).start()

        def superstep(k2, carry):
            chunk(0, k2 * 2, k2 * 2)
            chunk(1, k2 * 2 + 1, k2 * 2 + 1)
            return carry

        lax.fori_loop(0, nst // 2, superstep, 0)
        if nst % 2:
            chunk(0, nst - 1, nst - 1)
        out_copy(nst % 2, nst - 2 + (nst % 2)).wait()
        out_copy(1 - nst % 2, nst - 1).wait()

    return pl.kernel(
        body,
        out_type=jax.ShapeDtypeStruct((_VOCAB, nst, nbt, 8, 128),
                                      jnp.float32),
        mesh=mesh,
        scratch_types=[
            pltpu.VMEM((tiles_w, 8, 128), jnp.int32),
            pltpu.VMEM((tiles_w, 8, 128), jnp.int32),
            pltpu.VMEM((_VOCAB, tiles_w, 8, 128), jnp.float32),
            pltpu.VMEM((_VOCAB, tiles_w, 8, 128), jnp.float32),
            pltpu.VMEM((_VOCAB * _VOCAB,), jnp.float32),
            pltpu.SemaphoreType.DMA,
            pltpu.SemaphoreType.DMA,
            pltpu.SemaphoreType.DMA,
            pltpu.SemaphoreType.DMA,
        ],
        compiler_params=pltpu.CompilerParams(
            use_tc_tiling_on_sc=False, needs_layout_passes=False,
        ),
    )


@jax.jit
def kernel(input_ids, embed_weight, proj_weight, proj_bias):
    batch, seq = input_ids.shape
    nst, nbt = seq // 8, batch // 128
    tab = _fused_table_t(embed_weight, proj_weight, proj_bias)
    ids4 = input_ids.T.reshape(nst, 8, nbt, 128).transpose(0, 2, 1, 3)
    out5 = _make_lookup(nst, nbt)(ids4, tab.reshape(_VOCAB * _VOCAB))
    out3 = out5.transpose(0, 1, 3, 2, 4).reshape(_VOCAB, seq, batch)
    return out3.transpose(2, 1, 0)


# speedup vs baseline: 40.6245x; 1.0048x over previous
import jax
import jax.numpy as jnp
from jax import lax
from jax.experimental import pallas as pl
from jax.experimental.pallas import tpu as pltpu
from jax.experimental.pallas import tpu_sc as plsc

_VOCAB = 12
_NC = 2
_NS = 16
_NW = _NC * _NS


def _tab_body(e_ref, p_ref, b_ref, t_ref):
    t_ref[...] = lax.dot_general(
        p_ref[...], e_ref[...], (((1,), (1,)), ((), ())),
        preferred_element_type=jnp.float32,
    ) + b_ref[...]


def _fused_table_t(embed_weight, proj_weight, proj_bias):
    return pl.pallas_call(
        _tab_body,
        out_shape=jax.ShapeDtypeStruct((_VOCAB, _VOCAB), jnp.float32),
    )(embed_weight, proj_weight, proj_bias.reshape(_VOCAB, 1))


def _make_lookup(nst, nbt):
    tiles_w = nbt // _NW
    assert tiles_w * _NW == nbt
    nvec = tiles_w * 8 * 128 // 16

    mesh = plsc.VectorSubcoreMesh(
        core_axis_name="c", subcore_axis_name="s",
        num_cores=_NC, num_subcores=_NS,
    )

    def body(ids_hbm, tab_hbm, out_hbm,
             idx0, idx1, out0, out1, tab_v,
             sem_i0, sem_i1, sem_o0, sem_o1):
        wid = lax.axis_index("s") * _NC + lax.axis_index("c")
        bt0 = wid * tiles_w
        idx_bufs = (idx0, idx1)
        out_bufs = (out0, out1)
        sem_i = (sem_i0, sem_i1)
        sem_o = (sem_o0, sem_o1)

        pltpu.sync_copy(tab_hbm, tab_v)

        def ids_copy(b, c):
            return pltpu.make_async_copy(
                ids_hbm.at[c, pl.ds(bt0, tiles_w)],
                idx_bufs[b], sem_i[b],
            )

        def out_copy(b, c):
            return pltpu.make_async_copy(
                out_bufs[b],
                out_hbm.at[:, c, pl.ds(bt0, tiles_w)],
                sem_o[b],
            )

        ids_copy(0, 0).start()
        ids_copy(1, 1).start()

        def chunk(b, c, k):
            idx_v = idx_bufs[b]
            out_v = out_bufs[b]
            ids_copy(b, c).wait()

            @pl.when(k > 1)
            def _drain():
                out_copy(b, c).wait()

            @plsc.parallel_loop(0, nvec, unroll=4)
            def vstep(j):
                t = j // (8 * 8)
                rem = j % (8 * 8)
                r = rem // 8
                col = (rem % 8) * 16
                ids16 = idx_v[t, r, pl.ds(col, 16)]
                for v in range(_VOCAB):
                    out_v[v, t, r, pl.ds(col, 16)] = plsc.load_gather(
                        tab_v, [ids16 + v * _VOCAB])
                return None

            @pl.when(k < nst - 2)
            def _prefetch():
                ids_copy(b, c + 2).start()

            out_copy(b, c).start()

        def superstep(k2, carry):
            chunk(0, k2 * 2, k2 * 2)
            chunk(1, k2 * 2 + 1, k2 * 2 + 1)
            return carry

        lax.fori_loop(0, nst // 2, superstep, 0)
        if nst % 2:
            chunk(0, nst - 1, nst - 1)
        out_copy(nst % 2, nst - 2 + (nst % 2)).wait()
        out_copy(1 - nst % 2, nst - 1).wait()

    return pl.kernel(
        body,
        out_type=jax.ShapeDtypeStruct((_VOCAB, nst, nbt, 8, 128),
                                      jnp.float32),
        mesh=mesh,
        scratch_types=[
            pltpu.VMEM((tiles_w, 8, 128), jnp.int32),
            pltpu.VMEM((tiles_w, 8, 128), jnp.int32),
            pltpu.VMEM((_VOCAB, tiles_w, 8, 128), jnp.float32),
            pltpu.VMEM((_VOCAB, tiles_w, 8, 128), jnp.float32),
            pltpu.VMEM((_VOCAB * _VOCAB,), jnp.float32),
            pltpu.SemaphoreType.DMA,
            pltpu.SemaphoreType.DMA,
            pltpu.SemaphoreType.DMA,
            pltpu.SemaphoreType.DMA,
        ],
        compiler_params=pltpu.CompilerParams(
            use_tc_tiling_on_sc=False, needs_layout_passes=False,
        ),
    )


@jax.jit
def kernel(input_ids, embed_weight, proj_weight, proj_bias):
    batch, seq = input_ids.shape
    nst, nbt = seq // 8, batch // 128
    tab = _fused_table_t(embed_weight, proj_weight, proj_bias)
    ids4 = input_ids.T.reshape(nst, 8, nbt, 128).transpose(0, 2, 1, 3)
    out5 = _make_lookup(nst, nbt)(ids4, tab.reshape(_VOCAB * _VOCAB))
    out3 = out5.transpose(0, 1, 3, 2, 4).reshape(_VOCAB, seq, batch)
    return out3.transpose(2, 1, 0)


# speedup vs baseline: 40.6370x; 1.0003x over previous
import jax
import jax.numpy as jnp
from jax import lax
from jax.experimental import pallas as pl
from jax.experimental.pallas import tpu as pltpu
from jax.experimental.pallas import tpu_sc as plsc

_VOCAB = 12
_NC = 2
_NS = 16
_NW = _NC * _NS


def _tab_body(e_ref, p_ref, b_ref, t_ref):
    t_ref[...] = lax.dot_general(
        p_ref[...], e_ref[...], (((1,), (1,)), ((), ())),
        preferred_element_type=jnp.float32,
    ) + b_ref[...]


def _fused_table_t(embed_weight, proj_weight, proj_bias):
    return pl.pallas_call(
        _tab_body,
        out_shape=jax.ShapeDtypeStruct((_VOCAB, _VOCAB), jnp.float32),
    )(embed_weight, proj_weight, proj_bias.reshape(_VOCAB, 1))


def _make_lookup(nst, nbt):
    tiles_w = nbt // _NW
    assert tiles_w * _NW == nbt
    nvec = tiles_w * 8 * 128 // 16

    mesh = plsc.VectorSubcoreMesh(
        core_axis_name="c", subcore_axis_name="s",
        num_cores=_NC, num_subcores=_NS,
    )

    def body(ids_hbm, tab_hbm, out_hbm,
             idx0, idx1, out0, out1, tab_v,
             sem_i0, sem_i1, sem_o0, sem_o1):
        wid = lax.axis_index("s") * _NC + lax.axis_index("c")
        bt0 = wid * tiles_w
        idx_bufs = (idx0, idx1)
        out_bufs = (out0, out1)
        sem_i = (sem_i0, sem_i1)
        sem_o = (sem_o0, sem_o1)

        def ids_copy(b, c):
            return pltpu.make_async_copy(
                ids_hbm.at[c, pl.ds(bt0, tiles_w)],
                idx_bufs[b], sem_i[b],
            )

        def out_copy(b, c):
            return pltpu.make_async_copy(
                out_bufs[b],
                out_hbm.at[:, c, pl.ds(bt0, tiles_w)],
                sem_o[b],
            )

        ids_copy(0, 0).start()
        ids_copy(1, 1).start()

        pltpu.sync_copy(tab_hbm, tab_v)

        def chunk(b, c, k):
            idx_v = idx_bufs[b]
            out_v = out_bufs[b]
            ids_copy(b, c).wait()

            @pl.when(k > 1)
            def _drain():
                out_copy(b, c).wait()

            @plsc.parallel_loop(0, nvec, unroll=4)
            def vstep(j):
                t = j // (8 * 8)
                rem = j % (8 * 8)
                r = rem // 8
                col = (rem % 8) * 16
                ids16 = idx_v[t, r, pl.ds(col, 16)]
                for v in range(_VOCAB):
                    out_v[v, t, r, pl.ds(col, 16)] = plsc.load_gather(
                        tab_v, [ids16 + v * _VOCAB])
                return None

            @pl.when(k < nst - 2)
            def _prefetch():
                ids_copy(b, c + 2).start()

            out_copy(b, c).start()

        def superstep(k2, carry):
            chunk(0, k2 * 2, k2 * 2)
            chunk(1, k2 * 2 + 1, k2 * 2 + 1)
            return carry

        lax.fori_loop(0, nst // 2, superstep, 0)
        if nst % 2:
            chunk(0, nst - 1, nst - 1)
        out_copy(nst % 2, nst - 2 + (nst % 2)).wait()
        out_copy(1 - nst % 2, nst - 1).wait()

    return pl.kernel(
        body,
        out_type=jax.ShapeDtypeStruct((_VOCAB, nst, nbt, 8, 128),
                                      jnp.float32),
        mesh=mesh,
        scratch_types=[
            pltpu.VMEM((tiles_w, 8, 128), jnp.int32),
            pltpu.VMEM((tiles_w, 8, 128), jnp.int32),
            pltpu.VMEM((_VOCAB, tiles_w, 8, 128), jnp.float32),
            pltpu.VMEM((_VOCAB, tiles_w, 8, 128), jnp.float32),
            pltpu.VMEM((_VOCAB * _VOCAB,), jnp.float32),
            pltpu.SemaphoreType.DMA,
            pltpu.SemaphoreType.DMA,
            pltpu.SemaphoreType.DMA,
            pltpu.SemaphoreType.DMA,
        ],
        compiler_params=pltpu.CompilerParams(
            use_tc_tiling_on_sc=False, needs_layout_passes=False,
        ),
    )


@jax.jit
def kernel(input_ids, embed_weight, proj_weight, proj_bias):
    batch, seq = input_ids.shape
    nst, nbt = seq // 8, batch // 128
    tab = _fused_table_t(embed_weight, proj_weight, proj_bias)
    ids4 = input_ids.T.reshape(nst, 8, nbt, 128).transpose(0, 2, 1, 3)
    out5 = _make_lookup(nst, nbt)(ids4, tab.reshape(_VOCAB * _VOCAB))
    out3 = out5.transpose(0, 1, 3, 2, 4).reshape(_VOCAB, seq, batch)
    return out3.transpose(2, 1, 0)


# speedup vs baseline: 41.5580x; 1.0227x over previous
import jax
import jax.numpy as jnp
from jax import lax
from jax.experimental import pallas as pl
from jax.experimental.pallas import tpu as pltpu
from jax.experimental.pallas import tpu_sc as plsc

_VOCAB = 12
_NC = 2
_NS = 16
_NW = _NC * _NS


def _make_lookup(nst, nbt):
    tiles_w = nbt // _NW
    assert tiles_w * _NW == nbt
    nvec = tiles_w * 8 * 128 // 16

    mesh = plsc.VectorSubcoreMesh(
        core_axis_name="c", subcore_axis_name="s",
        num_cores=_NC, num_subcores=_NS,
    )

    def body(ids_hbm, e_hbm, p_hbm, b_hbm, out_hbm,
             idx0, idx1, out0, out1, e_v, p_v, bias_v, tab_v,
             sem_i0, sem_i1, sem_o0, sem_o1):
        wid = lax.axis_index("s") * _NC + lax.axis_index("c")
        bt0 = wid * tiles_w
        idx_bufs = (idx0, idx1)
        out_bufs = (out0, out1)
        sem_i = (sem_i0, sem_i1)
        sem_o = (sem_o0, sem_o1)

        def ids_copy(b, c):
            return pltpu.make_async_copy(
                ids_hbm.at[c, pl.ds(bt0, tiles_w)],
                idx_bufs[b], sem_i[b],
            )

        def out_copy(b, c):
            return pltpu.make_async_copy(
                out_bufs[b],
                out_hbm.at[:, c, pl.ds(bt0, tiles_w)],
                sem_o[b],
            )

        ids_copy(0, 0).start()
        ids_copy(1, 1).start()

        pltpu.sync_copy(e_hbm, e_v)
        pltpu.sync_copy(p_hbm, p_v)
        pltpu.sync_copy(b_hbm, bias_v)
        iota = lax.iota(jnp.int32, 16)
        for j in range(_VOCAB * _VOCAB // 16):
            c = iota + j * 16
            vv = c // _VOCAB
            ii = c % _VOCAB
            acc = plsc.load_gather(bias_v, [vv])
            for d in range(8):
                dd = iota * 0 + d
                acc = acc + (plsc.load_gather(p_v, [vv, dd])
                             * plsc.load_gather(e_v, [ii, dd]))
            tab_v[pl.ds(j * 16, 16)] = acc

        def chunk(b, c, k):
            idx_v = idx_bufs[b]
            out_v = out_bufs[b]
            ids_copy(b, c).wait()

            @pl.when(k > 1)
            def _drain():
                out_copy(b, c).wait()

            @plsc.parallel_loop(0, nvec, unroll=4)
            def vstep(j):
                t = j // (8 * 8)
                rem = j % (8 * 8)
                r = rem // 8
                col = (rem % 8) * 16
                ids16 = idx_v[t, r, pl.ds(col, 16)]
                for v in range(_VOCAB):
                    out_v[v, t, r, pl.ds(col, 16)] = plsc.load_gather(
                        tab_v, [ids16 + v * _VOCAB])
                return None

            @pl.when(k < nst - 2)
            def _prefetch():
                ids_copy(b, c + 2).start()

            out_copy(b, c).start()

        def superstep(k2, carry):
            chunk(0, k2 * 2, k2 * 2)
            chunk(1, k2 * 2 + 1, k2 * 2 + 1)
            return carry

        lax.fori_loop(0, nst // 2, superstep, 0)
        if nst % 2:
            chunk(0, nst - 1, nst - 1)
        out_copy(nst % 2, nst - 2 + (nst % 2)).wait()
        out_copy(1 - nst % 2, nst - 1).wait()

    return pl.kernel(
        body,
        out_type=jax.ShapeDtypeStruct((_VOCAB, nst, nbt, 8, 128),
                                      jnp.float32),
        mesh=mesh,
        scratch_types=[
            pltpu.VMEM((tiles_w, 8, 128), jnp.int32),
            pltpu.VMEM((tiles_w, 8, 128), jnp.int32),
            pltpu.VMEM((_VOCAB, tiles_w, 8, 128), jnp.float32),
            pltpu.VMEM((_VOCAB, tiles_w, 8, 128), jnp.float32),
            pltpu.VMEM((_VOCAB, 8), jnp.float32),
            pltpu.VMEM((_VOCAB, 8), jnp.float32),
            pltpu.VMEM((_VOCAB,), jnp.float32),
            pltpu.VMEM((_VOCAB * _VOCAB,), jnp.float32),
            pltpu.SemaphoreType.DMA,
            pltpu.SemaphoreType.DMA,
            pltpu.SemaphoreType.DMA,
            pltpu.SemaphoreType.DMA,
        ],
        compiler_params=pltpu.CompilerParams(
            use_tc_tiling_on_sc=False, needs_layout_passes=False,
        ),
    )


@jax.jit
def kernel(input_ids, embed_weight, proj_weight, proj_bias):
    batch, seq = input_ids.shape
    nst, nbt = seq // 8, batch // 128
    ids4 = input_ids.T.reshape(nst, 8, nbt, 128).transpose(0, 2, 1, 3)
    out5 = _make_lookup(nst, nbt)(
        ids4, embed_weight, proj_weight, proj_bias)
    out3 = out5.transpose(0, 1, 3, 2, 4).reshape(_VOCAB, seq, batch)
    return out3.transpose(2, 1, 0)
